# Initial kernel scaffold; baseline (speedup 1.0000x reference)
import jax
import jax.numpy as jnp
from jax.experimental import pallas as pl


def kernel(self_vectors, neighbor_vectors, masks, node_emb, W, b):
    raise NotImplementedError("write your pallas kernel here")


# speedup vs baseline: 1.2873x; 1.2873x over previous
import jax
import jax.numpy as jnp
from jax.experimental import pallas as pl


def _agg_body(self_ref, emb_ref, mask_ref, neigh_ref, w_ref, b_ref, out_ref):
    nv = neigh_ref[...]
    m = mask_ref[...]
    n = nv.shape[1]
    mean = jnp.sum(nv * m[:, :, None], axis=1) * (1.0 / n)
    w = w_ref[...]
    d = mean.shape[1]
    acc = jnp.dot(self_ref[...], w[0:d], preferred_element_type=jnp.float32)
    acc = acc + jnp.dot(mean, w[d:2 * d], preferred_element_type=jnp.float32)
    acc = acc + jnp.dot(emb_ref[...], w[2 * d:3 * d],
                        preferred_element_type=jnp.float32)
    out_ref[...] = jnp.maximum(acc + b_ref[...], 0.0)


def kernel(self_vectors, neighbor_vectors, masks, node_emb, W, b):
    B_, _, H_, D_ = self_vectors.shape
    N_ = neighbor_vectors.shape[2]
    O_ = W.shape[1]
    BH = B_ * H_
    sv = self_vectors.reshape(BH, D_)
    nv = neighbor_vectors.reshape(BH, N_, D_)
    mk = masks.reshape(BH, N_)
    ne = node_emb.reshape(BH, D_)
    b2 = b.reshape(1, O_)

    R = 1024
    grid = (BH // R,)
    out = pl.pallas_call(
        _agg_body,
        grid=grid,
        in_specs=[
            pl.BlockSpec((R, D_), lambda i: (i, 0)),
            pl.BlockSpec((R, D_), lambda i: (i, 0)),
            pl.BlockSpec((R, N_), lambda i: (i, 0)),
            pl.BlockSpec((R, N_, D_), lambda i: (i, 0, 0)),
            pl.BlockSpec((3 * D_, O_), lambda i: (0, 0)),
            pl.BlockSpec((1, O_), lambda i: (0, 0)),
        ],
        out_specs=pl.BlockSpec((R, O_), lambda i: (i, 0)),
        out_shape=jax.ShapeDtypeStruct((BH, O_), jnp.float32),
    )(sv, ne, mk, nv, W, b2)
    return out.reshape(B_, 1, H_, O_)
